# baseline (speedup 1.0000x reference)
import functools

import jax
import jax.numpy as jnp
from jax import lax
from jax.experimental import pallas as pl
from jax.experimental.pallas import tpu as pltpu
from jax.experimental.pallas import tpu_sc as plsc

B = 2; N = 4096; FEAT = 128; KOD = 128; K = 15; NL = 32
CONV_R = 0.125; KERN_R = 0.06; EPS = 1e-12

_NC = 2
_NS = 16
_NW = _NC * _NS
_ROWS = B * N * NL
_RPW = _ROWS // _NW
_GPW = _RPW // 128
_CHUNK = 512
_NCHUNK = _RPW // _CHUNK
_PD = 128


def _leaky(x):
    return jnp.where(x >= 0, x, 0.1 * x)


def _ball_query(pts):
    sq = jnp.sum(pts ** 2, -1)
    d2 = sq[:, :, None] + sq[:, None, :] - 2.0 * jnp.einsum('bnc,bmc->bnm', pts, pts)
    base = jnp.arange(pts.shape[1], dtype=jnp.int32)[None, None, :]
    idx = jnp.where(d2 > CONV_R ** 2, pts.shape[1], base).astype(jnp.int32)
    return jnp.sort(idx, axis=-1)[:, :, :NL]


def _sc_gather_body(f0_hbm, f1_hbm, pts_hbm, gidx_hbm, nf0_hbm, nf1_hbm, pos_hbm,
                    idx_v, rows_v, sem):
    wid = lax.axis_index("s") * _NC + lax.axis_index("c")
    base_rows = wid * _RPW
    pltpu.sync_copy(gidx_hbm.at[pl.ds(wid * _GPW, _GPW)], idx_v)

    def feat_loop(tab_hbm, out_hbm):
        def body(c, carry):
            das = []
            for u in range(4):
                g = c * 4 + u
                das.append(pltpu.async_copy(
                    tab_hbm.at[idx_v.at[g]],
                    rows_v.at[pl.ds(u * 128, 128)], sem))
            for da in das:
                da.wait()
            pltpu.sync_copy(rows_v, out_hbm.at[pl.ds(base_rows + c * _CHUNK, _CHUNK)])
            return carry
        lax.fori_loop(0, _NCHUNK, body, 0)

    feat_loop(f0_hbm, nf0_hbm)
    feat_loop(f1_hbm, nf1_hbm)

    def pos_body(c, carry):
        das = []
        for u in range(4):
            g = c * 4 + u
            das.append(pltpu.async_copy(
                pts_hbm.at[idx_v.at[g]],
                rows_v.at[pl.ds(u * 128, 128)], sem))
        for da in das:
            da.wait()
        pltpu.sync_copy(rows_v, pos_hbm.at[pl.ds(base_rows + c * _CHUNK, _CHUNK)])
        return carry
    lax.fori_loop(0, _NCHUNK, pos_body, 0)


_sc_gather = functools.partial(
    pl.kernel,
    out_type=(
        jax.ShapeDtypeStruct((_ROWS, FEAT), jnp.float32),
        jax.ShapeDtypeStruct((_ROWS, FEAT), jnp.float32),
        jax.ShapeDtypeStruct((_ROWS, _PD), jnp.float32),
    ),
    mesh=plsc.VectorSubcoreMesh(core_axis_name="c", subcore_axis_name="s"),
    scratch_types=[
        pltpu.VMEM((_GPW, 128), jnp.int32),
        pltpu.VMEM((_CHUNK, FEAT), jnp.float32),
        pltpu.SemaphoreType.DMA,
    ],
)(_sc_gather_body)


def _final_kernel(x_ref, w_ref, b_ref, o_ref):
    o_ref[...] = _leaky(
        jnp.dot(x_ref[...], w_ref[...], preferred_element_type=jnp.float32)
        + b_ref[...][None, :])


def kernel(support_points, support_features, weights, deformed_weights, W_db, b_db, W_de, b_de, W_doff, b_doff, W_b, b_b, W_e, b_e, offset_bias, kernel_points):
    kp = kernel_points
    pts = support_points.transpose(0, 2, 1)
    feats = support_features.transpose(0, 2, 1)
    idx = _ball_query(pts)

    f0 = feats @ W_db.T + b_db
    f1 = feats @ W_b.T + b_b
    zrow = jnp.zeros((B, 1, FEAT), jnp.float32)
    f0p = jnp.concatenate([f0, zrow], axis=1).reshape(B * (N + 1), FEAT)
    f1p = jnp.concatenate([f1, zrow], axis=1).reshape(B * (N + 1), FEAT)
    ptsp = jnp.concatenate([pts, jnp.full((B, 1, 3), 1e6, jnp.float32)], axis=1)
    ptsp = jnp.pad(ptsp, ((0, 0), (0, 0), (0, _PD - 3))).reshape(B * (N + 1), _PD)
    gidx = (idx + (jnp.arange(B, dtype=jnp.int32) * (N + 1))[:, None, None]).reshape(
        _ROWS // 128, 128)

    nf0_flat, nf1_flat, pos_flat = _sc_gather(f0p, f1p, ptsp, gidx)
    nf0 = nf0_flat.reshape(B, N, NL, FEAT)
    nf1 = nf1_flat.reshape(B, N, NL, FEAT)
    nbr_pos = pos_flat.reshape(B, N, NL, _PD)[..., :3]

    rel = nbr_pos - pts[:, :, None, :]
    d20 = jnp.sum((rel[:, :, :, None, :] - kp[None, None, None, :, :]) ** 2, -1)
    infl0 = jnp.maximum(0.0, 1.0 - jnp.sqrt(d20 + EPS) / KERN_R)
    pk0 = jnp.einsum('bnsk,bnsc->bnkc', infl0, nf0)
    agg0 = _leaky(jnp.einsum('bnkc,kcd->bnd', pk0, deformed_weights))
    offf = _leaky(agg0 @ W_de.T + b_de)
    off = (offf @ W_doff.T + b_doff + offset_bias).reshape(B, N, K, 3)
    dkp = kp[None, None, :, :] + off
    d21 = jnp.sum((rel[:, :, :, None, :] - dkp[:, :, None, :, :]) ** 2, -1)
    infl1 = jnp.maximum(0.0, 1.0 - jnp.sqrt(d21 + EPS) / KERN_R)
    pk1 = jnp.einsum('bnsk,bnsc->bnkc', infl1, nf1)
    out = _leaky(jnp.einsum('bnkc,kcd->bnd', pk1, weights))
    fin = pl.pallas_call(
        _final_kernel,
        out_shape=jax.ShapeDtypeStruct((B * N, KOD), jnp.float32),
        grid=(B * N // 512,),
        in_specs=[
            pl.BlockSpec((512, KOD), lambda i: (i, 0)),
            pl.BlockSpec((KOD, KOD), lambda i: (0, 0)),
            pl.BlockSpec((KOD,), lambda i: (0,)),
        ],
        out_specs=pl.BlockSpec((512, KOD), lambda i: (i, 0)),
    )(out.reshape(B * N, KOD), W_e.T, b_e)
    return fin.reshape(B, N, KOD)


# speedup vs baseline: 2.4295x; 2.4295x over previous
import functools

import jax
import jax.numpy as jnp
from jax import lax
from jax.experimental import pallas as pl
from jax.experimental.pallas import tpu as pltpu
from jax.experimental.pallas import tpu_sc as plsc

B = 2; N = 4096; FEAT = 128; KOD = 128; K = 15; NL = 32
CONV_R = 0.125; KERN_R = 0.06; EPS = 1e-12

_NC = 2
_NS = 16
_NW = _NC * _NS
_ROWS = B * N * NL
_RPW = _ROWS // _NW
_GPW = _RPW // 128
_CHUNK = 512
_NCHUNK = _RPW // _CHUNK
_PD = 128


def _leaky(x):
    return jnp.where(x >= 0, x, 0.1 * x)


_BQ_BLK = 256


def _bq_body(ptsT_ref, pts_ref, out_ref):
    pr = ptsT_ref[0]
    xr, yr, zr = pr[:, 0:1], pr[:, 1:2], pr[:, 2:3]
    pa = pts_ref[0]
    x, y, z = pa[0:1, :], pa[1:2, :], pa[2:3, :]
    sqr = xr * xr + yr * yr + zr * zr
    sq = x * x + y * y + z * z
    def _r(v):
        return v.astype(jnp.bfloat16).astype(jnp.float32)
    dot = _r(xr) * _r(x) + _r(yr) * _r(y) + _r(zr) * _r(z)
    d2 = sqr + sq - 2.0 * dot
    iota = lax.broadcasted_iota(jnp.int32, (_BQ_BLK, N), 1).astype(jnp.float32)
    cand = jnp.where(d2 <= CONV_R * CONV_R, iota, float(N))
    cols = []
    for _ in range(NL):
        m = jnp.min(cand, axis=1, keepdims=True)
        cols.append(m)
        cand = jnp.where(cand == m, float(N), cand)
    out_ref[0] = jnp.concatenate(cols, axis=1).astype(jnp.int32)


def _ball_query(pts):
    ptsT = pts
    ptsC = pts.transpose(0, 2, 1)
    return pl.pallas_call(
        _bq_body,
        out_shape=jax.ShapeDtypeStruct((B, N, NL), jnp.int32),
        grid=(B, N // _BQ_BLK),
        in_specs=[
            pl.BlockSpec((1, _BQ_BLK, 3), lambda b, i: (b, i, 0)),
            pl.BlockSpec((1, 3, N), lambda b, i: (b, 0, 0)),
        ],
        out_specs=pl.BlockSpec((1, _BQ_BLK, NL), lambda b, i: (b, i, 0)),
    )(ptsT, ptsC)


def _sc_gather_body(f0_hbm, f1_hbm, pts_hbm, gidx_hbm, nf0_hbm, nf1_hbm, pos_hbm,
                    idx_v, rows_v, sem):
    wid = lax.axis_index("s") * _NC + lax.axis_index("c")
    base_rows = wid * _RPW
    pltpu.sync_copy(gidx_hbm.at[pl.ds(wid * _GPW, _GPW)], idx_v)

    def feat_loop(tab_hbm, out_hbm):
        def body(c, carry):
            das = []
            for u in range(4):
                g = c * 4 + u
                das.append(pltpu.async_copy(
                    tab_hbm.at[idx_v.at[g]],
                    rows_v.at[pl.ds(u * 128, 128)], sem))
            for da in das:
                da.wait()
            pltpu.sync_copy(rows_v, out_hbm.at[pl.ds(base_rows + c * _CHUNK, _CHUNK)])
            return carry
        lax.fori_loop(0, _NCHUNK, body, 0)

    feat_loop(f0_hbm, nf0_hbm)
    feat_loop(f1_hbm, nf1_hbm)

    def pos_body(c, carry):
        das = []
        for u in range(4):
            g = c * 4 + u
            das.append(pltpu.async_copy(
                pts_hbm.at[idx_v.at[g]],
                rows_v.at[pl.ds(u * 128, 128)], sem))
        for da in das:
            da.wait()
        pltpu.sync_copy(rows_v, pos_hbm.at[pl.ds(base_rows + c * _CHUNK, _CHUNK)])
        return carry
    lax.fori_loop(0, _NCHUNK, pos_body, 0)


@functools.lru_cache(maxsize=None)
def _sc_gather():
    return functools.partial(
        pl.kernel,
        out_type=(
            jax.ShapeDtypeStruct((_ROWS, FEAT), jnp.float32),
            jax.ShapeDtypeStruct((_ROWS, FEAT), jnp.float32),
            jax.ShapeDtypeStruct((_ROWS, _PD), jnp.float32),
        ),
        mesh=plsc.VectorSubcoreMesh(core_axis_name="c", subcore_axis_name="s",
                                    num_cores=_NC, num_subcores=_NS),
        scratch_types=[
            pltpu.VMEM((_GPW, 128), jnp.int32),
            pltpu.VMEM((_CHUNK, FEAT), jnp.float32),
            pltpu.SemaphoreType.DMA,
        ],
    )(_sc_gather_body)


def _final_kernel(x_ref, w_ref, b_ref, o_ref):
    o_ref[...] = _leaky(
        jnp.dot(x_ref[...], w_ref[...], preferred_element_type=jnp.float32)
        + b_ref[...][None, :])


def kernel(support_points, support_features, weights, deformed_weights, W_db, b_db, W_de, b_de, W_doff, b_doff, W_b, b_b, W_e, b_e, offset_bias, kernel_points):
    kp = kernel_points
    pts = support_points.transpose(0, 2, 1)
    feats = support_features.transpose(0, 2, 1)
    idx = _ball_query(pts)

    f0 = feats @ W_db.T + b_db
    f1 = feats @ W_b.T + b_b
    zrow = jnp.zeros((B, 1, FEAT), jnp.float32)
    f0p = jnp.concatenate([f0, zrow], axis=1).reshape(B * (N + 1), FEAT)
    f1p = jnp.concatenate([f1, zrow], axis=1).reshape(B * (N + 1), FEAT)
    ptsp = jnp.concatenate([pts, jnp.full((B, 1, 3), 1e6, jnp.float32)], axis=1)
    ptsp = jnp.pad(ptsp, ((0, 0), (0, 0), (0, _PD - 3))).reshape(B * (N + 1), _PD)
    gidx = (idx + (jnp.arange(B, dtype=jnp.int32) * (N + 1))[:, None, None]).reshape(
        _ROWS // 128, 128)

    nf0_flat, nf1_flat, pos_flat = _sc_gather()(f0p, f1p, ptsp, gidx)
    nf0 = nf0_flat.reshape(B, N, NL, FEAT)
    nf1 = nf1_flat.reshape(B, N, NL, FEAT)
    nbr_pos = pos_flat.reshape(B, N, NL, _PD)[..., :3]

    rel = nbr_pos - pts[:, :, None, :]
    d20 = jnp.sum((rel[:, :, :, None, :] - kp[None, None, None, :, :]) ** 2, -1)
    infl0 = jnp.maximum(0.0, 1.0 - jnp.sqrt(d20 + EPS) / KERN_R)
    pk0 = jnp.einsum('bnsk,bnsc->bnkc', infl0, nf0)
    agg0 = _leaky(jnp.einsum('bnkc,kcd->bnd', pk0, deformed_weights))
    offf = _leaky(agg0 @ W_de.T + b_de)
    off = (offf @ W_doff.T + b_doff + offset_bias).reshape(B, N, K, 3)
    dkp = kp[None, None, :, :] + off
    d21 = jnp.sum((rel[:, :, :, None, :] - dkp[:, :, None, :, :]) ** 2, -1)
    infl1 = jnp.maximum(0.0, 1.0 - jnp.sqrt(d21 + EPS) / KERN_R)
    pk1 = jnp.einsum('bnsk,bnsc->bnkc', infl1, nf1)
    out = _leaky(jnp.einsum('bnkc,kcd->bnd', pk1, weights))
    fin = pl.pallas_call(
        _final_kernel,
        out_shape=jax.ShapeDtypeStruct((B * N, KOD), jnp.float32),
        grid=(B * N // 512,),
        in_specs=[
            pl.BlockSpec((512, KOD), lambda i: (i, 0)),
            pl.BlockSpec((KOD, KOD), lambda i: (0, 0)),
            pl.BlockSpec((KOD,), lambda i: (0,)),
        ],
        out_specs=pl.BlockSpec((512, KOD), lambda i: (i, 0)),
    )(out.reshape(B * N, KOD), W_e.T, b_e)
    return fin.reshape(B, N, KOD)


# speedup vs baseline: 2.4371x; 1.0031x over previous
import functools

import jax
import jax.numpy as jnp
from jax import lax
from jax.experimental import pallas as pl
from jax.experimental.pallas import tpu as pltpu
from jax.experimental.pallas import tpu_sc as plsc

B = 2; N = 4096; FEAT = 128; KOD = 128; K = 15; NL = 32
CONV_R = 0.125; KERN_R = 0.06; EPS = 1e-12

_NC = 2
_NS = 16
_NW = _NC * _NS
_ROWS = B * N * NL
_RPW = _ROWS // _NW
_GPW = _RPW // 128
_CHUNK = 512
_NCHUNK = _RPW // _CHUNK
_PD = 128


def _leaky(x):
    return jnp.where(x >= 0, x, 0.1 * x)


_BQ_BLK = 256


def _bq_body(ptsT_ref, pts_ref, out_ref):
    pr = ptsT_ref[0]
    xr, yr, zr = pr[:, 0:1], pr[:, 1:2], pr[:, 2:3]
    pa = pts_ref[0]
    x, y, z = pa[0:1, :], pa[1:2, :], pa[2:3, :]
    sqr = xr * xr + yr * yr + zr * zr
    sq = x * x + y * y + z * z
    def _r(v):
        return v.astype(jnp.bfloat16).astype(jnp.float32)
    dot = _r(xr) * _r(x) + _r(yr) * _r(y) + _r(zr) * _r(z)
    d2 = sqr + sq - 2.0 * dot
    iota = lax.broadcasted_iota(jnp.int32, (_BQ_BLK, N), 1).astype(jnp.float32)
    cand = jnp.where(d2 <= CONV_R * CONV_R, iota, float(N))
    cols = []
    for _ in range(NL):
        m = jnp.min(cand, axis=1, keepdims=True)
        cols.append(m)
        cand = jnp.where(cand == m, float(N), cand)
    out_ref[0] = jnp.concatenate(cols, axis=1).astype(jnp.int32)


def _ball_query(pts):
    ptsT = pts
    ptsC = pts.transpose(0, 2, 1)
    return pl.pallas_call(
        _bq_body,
        out_shape=jax.ShapeDtypeStruct((B, N, NL), jnp.int32),
        grid=(B, N // _BQ_BLK),
        in_specs=[
            pl.BlockSpec((1, _BQ_BLK, 3), lambda b, i: (b, i, 0)),
            pl.BlockSpec((1, 3, N), lambda b, i: (b, 0, 0)),
        ],
        out_specs=pl.BlockSpec((1, _BQ_BLK, NL), lambda b, i: (b, i, 0)),
    )(ptsT, ptsC)


def _sc_gather_body(f0_hbm, f1_hbm, pts_hbm, gidx_hbm,
                    nf0_hbm, nf1_hbm, pos_hbm,
                    idx_v, bufa_v, bufb_v, sema, semb):
    wid = lax.axis_index("s") * _NC + lax.axis_index("c")
    pltpu.sync_copy(gidx_hbm.at[pl.ds(wid * _GPW, _GPW)], idx_v)

    def feat_loop(tab_hbm, out_hbm):
        def fire(c, buf, sem):
            for h in range(2):
                pltpu.async_copy(tab_hbm.at[idx_v.at[c * 2 + h]], buf.at[h], sem)

        def drain(buf, sem):
            pltpu.make_async_copy(out_hbm.at[pl.ds(0, 2)], buf, sem).wait()

        fire(0, bufa_v, sema)

        def body(i, carry):
            c0 = 2 * i
            fire(c0 + 1, bufb_v, semb)
            drain(bufa_v, sema)
            pltpu.sync_copy(bufa_v, out_hbm.at[pl.ds(wid * 2 * _NCH + c0 * 2, 2)])

            @pl.when(i < _NCH // 2 - 1)
            def _():
                fire(c0 + 2, bufa_v, sema)
            drain(bufb_v, semb)
            pltpu.sync_copy(bufb_v, out_hbm.at[pl.ds(wid * 2 * _NCH + c0 * 2 + 2, 2)])
            return carry
        lax.fori_loop(0, _NCH // 2, body, 0)

    feat_loop(f0_hbm, nf0_hbm)
    feat_loop(f1_hbm, nf1_hbm)
    feat_loop(pts_hbm, pos_hbm)


_NCH = 32


@functools.lru_cache(maxsize=None)
def _sc_gather():
    return functools.partial(
        pl.kernel,
        out_type=(
            jax.ShapeDtypeStruct((_ROWS // 128, 128, FEAT), jnp.float32),
            jax.ShapeDtypeStruct((_ROWS // 128, 128, FEAT), jnp.float32),
            jax.ShapeDtypeStruct((_ROWS // 128, 128, FEAT), jnp.float32),
        ),
        mesh=plsc.VectorSubcoreMesh(core_axis_name="c", subcore_axis_name="s",
                                    num_cores=_NC, num_subcores=_NS),
        scratch_types=[
            pltpu.VMEM((_GPW, 128), jnp.int32),
            pltpu.VMEM((2, 128, FEAT), jnp.float32),
            pltpu.VMEM((2, 128, FEAT), jnp.float32),
            pltpu.SemaphoreType.DMA,
            pltpu.SemaphoreType.DMA,
        ],
    )(_sc_gather_body)


def _final_kernel(x_ref, w_ref, b_ref, o_ref):
    o_ref[...] = _leaky(
        jnp.dot(x_ref[...], w_ref[...], preferred_element_type=jnp.float32)
        + b_ref[...][None, :])


def kernel(support_points, support_features, weights, deformed_weights, W_db, b_db, W_de, b_de, W_doff, b_doff, W_b, b_b, W_e, b_e, offset_bias, kernel_points):
    kp = kernel_points
    pts = support_points.transpose(0, 2, 1)
    feats = support_features.transpose(0, 2, 1)
    idx = _ball_query(pts)

    f0 = feats @ W_db.T + b_db
    f1 = feats @ W_b.T + b_b
    zrow = jnp.zeros((B, 1, FEAT), jnp.float32)
    f0p = jnp.concatenate([f0, zrow], axis=1).reshape(B * (N + 1), FEAT)
    f1p = jnp.concatenate([f1, zrow], axis=1).reshape(B * (N + 1), FEAT)
    ptsp = jnp.concatenate([pts, jnp.full((B, 1, 3), 1e6, jnp.float32)], axis=1)
    ptsp = jnp.pad(ptsp, ((0, 0), (0, 0), (0, FEAT - 3))).reshape(B * (N + 1), FEAT)
    gidx = (idx + (jnp.arange(B, dtype=jnp.int32) * (N + 1))[:, None, None]).reshape(
        _ROWS // 128, 128)

    nf0_flat, nf1_flat, pos_flat = _sc_gather()(f0p, f1p, ptsp, gidx)
    nf0 = nf0_flat.reshape(B, N, NL, FEAT)
    nf1 = nf1_flat.reshape(B, N, NL, FEAT)
    nbr_pos = pos_flat.reshape(B, N, NL, FEAT)[..., :3]

    rel = nbr_pos - pts[:, :, None, :]
    d20 = jnp.sum((rel[:, :, :, None, :] - kp[None, None, None, :, :]) ** 2, -1)
    infl0 = jnp.maximum(0.0, 1.0 - jnp.sqrt(d20 + EPS) / KERN_R)
    pk0 = jnp.einsum('bnsk,bnsc->bnkc', infl0, nf0)
    agg0 = _leaky(jnp.einsum('bnkc,kcd->bnd', pk0, deformed_weights))
    offf = _leaky(agg0 @ W_de.T + b_de)
    off = (offf @ W_doff.T + b_doff + offset_bias).reshape(B, N, K, 3)
    dkp = kp[None, None, :, :] + off
    d21 = jnp.sum((rel[:, :, :, None, :] - dkp[:, :, None, :, :]) ** 2, -1)
    infl1 = jnp.maximum(0.0, 1.0 - jnp.sqrt(d21 + EPS) / KERN_R)
    pk1 = jnp.einsum('bnsk,bnsc->bnkc', infl1, nf1)
    out = _leaky(jnp.einsum('bnkc,kcd->bnd', pk1, weights))
    fin = pl.pallas_call(
        _final_kernel,
        out_shape=jax.ShapeDtypeStruct((B * N, KOD), jnp.float32),
        grid=(B * N // 512,),
        in_specs=[
            pl.BlockSpec((512, KOD), lambda i: (i, 0)),
            pl.BlockSpec((KOD, KOD), lambda i: (0, 0)),
            pl.BlockSpec((KOD,), lambda i: (0,)),
        ],
        out_specs=pl.BlockSpec((512, KOD), lambda i: (i, 0)),
    )(out.reshape(B * N, KOD), W_e.T, b_e)
    return fin.reshape(B, N, KOD)
